# baseline (speedup 1.0000x reference)
import functools

import jax
import jax.numpy as jnp
from jax import lax
from jax.experimental import pallas as pl
from jax.experimental.pallas import tpu as pltpu
from jax.experimental.pallas import tpu_sc as plsc

_B = 16384
_D = 32
_CHUNK = 128
_LN2 = 0.6931471805599453


def _rsqrt16(x):
    i = lax.bitcast_convert_type(x, jnp.int32)
    one = jnp.full((16,), 1, jnp.int32)
    y = lax.bitcast_convert_type(
        jnp.full((16,), 0x5F3759DF, jnp.int32) - lax.shift_right_logical(i, one),
        jnp.float32)
    for _ in range(3):
        y = y * (1.5 - 0.5 * x * y * y)
    return y


def _ln16(x):
    f = lax.bitcast_convert_type(x, jnp.int32).astype(jnp.float32)
    y = f * (_LN2 / (1 << 23)) - (127.0 * _LN2 + 0.0298)
    for _ in range(3):
        y = y + x * jnp.exp(-y) - 1.0
    return y


def _sc_loss_parts(uid, ipid, inid, hid, rid, ptid, ntid, ut4, it4, et4, rt4):
    info = plsc.get_sparse_core_info()
    nc, ns = info.num_cores, info.num_subcores
    nw = nc * ns
    rpw = _B // nw
    nchunk = rpw // _CHUNK
    ngrp = _CHUNK // 16

    mesh = plsc.VectorSubcoreMesh(core_axis_name="c", subcore_axis_name="s")

    @functools.partial(
        pl.kernel,
        mesh=mesh,
        compiler_params=pltpu.CompilerParams(needs_layout_passes=False),
        out_type=jax.ShapeDtypeStruct((nw, 16, 16), jnp.float32),
        scratch_types=(
            [pltpu.VMEM((rpw,), jnp.int32)] * 7
            + [pltpu.VMEM((rpw,), jnp.int32)] * 4
            + [pltpu.VMEM((_CHUNK, 4 * _D), jnp.float32)] * 5
            + [pltpu.VMEM((16, 16), jnp.float32), pltpu.SemaphoreType.DMA]
        ),
    )
    def body(uid_h, ipid_h, inid_h, hid_h, rid_h, ptid_h, ntid_h,
             ut_h, it_h, et_h, rt_h, out_h,
             xu, xip, xin, xh, xr, xpt, xnt,
             q0, q1, q2, q3,
             b0, b1, b2, b3, b4, accv, sem):
        wid = lax.axis_index("s") * nc + lax.axis_index("c")
        sl = pl.ds(wid * rpw, rpw)
        pltpu.sync_copy(hid_h.at[sl], xh)
        pltpu.sync_copy(rid_h.at[sl], xr)
        pltpu.sync_copy(ptid_h.at[sl], xpt)
        pltpu.sync_copy(ntid_h.at[sl], xnt)
        pltpu.sync_copy(uid_h.at[sl], xu)
        pltpu.sync_copy(ipid_h.at[sl], xip)
        pltpu.sync_copy(inid_h.at[sl], xin)

        iota = lax.iota(jnp.int32, 16)
        zero = jnp.zeros((16,), jnp.float32)
        two = jnp.full((16,), 2, jnp.int32)
        three = jnp.full((16,), 3, jnp.int32)

        def shr2(src, dst):
            for k in range(rpw // 16):
                v = src[pl.ds(k * 16, 16)]
                dst[pl.ds(k * 16, 16)] = lax.shift_right_logical(v, two)

        shr2(xh, q0)
        shr2(xr, q1)
        shr2(xpt, q2)
        shr2(xnt, q3)

        def kg_chunk(c, kg_carry):
            cs = pl.ds(c * _CHUNK, _CHUNK)
            d0 = pltpu.async_copy(et_h.at[q0.at[cs]], b0, sem)
            d1 = pltpu.async_copy(rt_h.at[q1.at[cs]], b1, sem)
            d2 = pltpu.async_copy(et_h.at[q2.at[cs]], b2, sem)
            d3 = pltpu.async_copy(et_h.at[q3.at[cs]], b3, sem)
            d0.wait(); d1.wait(); d2.wait(); d3.wait()

            def kg_group(g, carry):
                a_h, a_r, a_p, a_n, c_hr, c_hp, c_hn, c_rp, c_rn = carry
                base = c * _CHUNK + g * 16
                row = g * 16 + iota
                ch = lax.shift_left(jnp.bitwise_and(xh[pl.ds(base, 16)], three), jnp.full((16,), 5, jnp.int32))
                cr = lax.shift_left(jnp.bitwise_and(xr[pl.ds(base, 16)], three), jnp.full((16,), 5, jnp.int32))
                cp = lax.shift_left(jnp.bitwise_and(xpt[pl.ds(base, 16)], three), jnp.full((16,), 5, jnp.int32))
                cn = lax.shift_left(jnp.bitwise_and(xnt[pl.ds(base, 16)], three), jnp.full((16,), 5, jnp.int32))
                sh = sr = sp = sn = hr = hp = hn = rp = rn = zero
                for d in range(_D):
                    dd = jnp.full((16,), d, jnp.int32)
                    hv = plsc.load_gather(b0, [row, ch + dd])
                    rv = plsc.load_gather(b1, [row, cr + dd])
                    pv = plsc.load_gather(b2, [row, cp + dd])
                    nv = plsc.load_gather(b3, [row, cn + dd])
                    sh = sh + hv * hv
                    sr = sr + rv * rv
                    sp = sp + pv * pv
                    sn = sn + nv * nv
                    hr = hr + hv * rv
                    hp = hp + hv * pv
                    hn = hn + hv * nv
                    rp = rp + rv * pv
                    rn = rn + rv * nv
                ih = _rsqrt16(jnp.maximum(sh, 1e-24))
                ir = _rsqrt16(jnp.maximum(sr, 1e-24))
                ip_ = _rsqrt16(jnp.maximum(sp, 1e-24))
                in_ = _rsqrt16(jnp.maximum(sn, 1e-24))
                return (a_h + sh * ih * ih, a_r + sr * ir * ir,
                        a_p + sp * ip_ * ip_, a_n + sn * in_ * in_,
                        c_hr + hr * ih * ir, c_hp + hp * ih * ip_,
                        c_hn + hn * ih * in_, c_rp + rp * ir * ip_,
                        c_rn + rn * ir * in_)

            return lax.fori_loop(0, ngrp, kg_group, kg_carry)

        kg_acc = lax.fori_loop(0, nchunk, kg_chunk, (zero,) * 9)

        shr2(xu, q0)
        shr2(xip, q1)
        shr2(xin, q2)

        def cf_chunk(c, cf_carry):
            cs = pl.ds(c * _CHUNK, _CHUNK)
            d0 = pltpu.async_copy(ut_h.at[q0.at[cs]], b0, sem)
            d1 = pltpu.async_copy(it_h.at[q1.at[cs]], b1, sem)
            d2 = pltpu.async_copy(et_h.at[q1.at[cs]], b2, sem)
            d3 = pltpu.async_copy(it_h.at[q2.at[cs]], b3, sem)
            d4 = pltpu.async_copy(et_h.at[q2.at[cs]], b4, sem)
            d0.wait(); d1.wait(); d2.wait(); d3.wait(); d4.wait()

            def cf_group(g, carry):
                slog, su, sp2, sn2 = carry
                base = c * _CHUNK + g * 16
                row = g * 16 + iota
                cu = lax.shift_left(jnp.bitwise_and(xu[pl.ds(base, 16)], three), jnp.full((16,), 5, jnp.int32))
                cp = lax.shift_left(jnp.bitwise_and(xip[pl.ds(base, 16)], three), jnp.full((16,), 5, jnp.int32))
                cn = lax.shift_left(jnp.bitwise_and(xin[pl.ds(base, 16)], three), jnp.full((16,), 5, jnp.int32))
                up = un = ru = rp_ = rn_ = zero
                for d in range(_D):
                    dd = jnp.full((16,), d, jnp.int32)
                    uv = plsc.load_gather(b0, [row, cu + dd])
                    ipv = plsc.load_gather(b1, [row, cp + dd])
                    epv = plsc.load_gather(b2, [row, cp + dd])
                    inv = plsc.load_gather(b3, [row, cn + dd])
                    env = plsc.load_gather(b4, [row, cn + dd])
                    pv = ipv + epv
                    nv = inv + env
                    up = up + uv * pv
                    un = un + uv * nv
                    ru = ru + uv * uv
                    rp_ = rp_ + pv * pv
                    rn_ = rn_ + nv * nv
                dlt = up - un
                sig = 1.0 / (1.0 + jnp.exp(-dlt))
                y = _ln16(sig + 1e-10)
                return (slog - y, su + ru, sp2 + rp_, sn2 + rn_)

            return lax.fori_loop(0, ngrp, cf_group, cf_carry)

        cf_acc = lax.fori_loop(0, nchunk, cf_chunk, (zero,) * 4)

        vals = list(kg_acc) + list(cf_acc)
        for k in range(13):
            accv[k, :] = vals[k]
        for k in range(13, 16):
            accv[k, :] = zero
        pltpu.sync_copy(accv, out_h.at[wid])

    return body(uid, ipid, inid, hid, rid, ptid, ntid, ut4, it4, et4, rt4)


def kernel(user_ids, item_pos_ids, item_neg_ids, h, r, pos_t, neg_t, is_train,
           user_table, item_table, entity_table, relation_table):
    del is_train
    i32 = jnp.int32
    parts = _sc_loss_parts(
        user_ids.astype(i32), item_pos_ids.astype(i32), item_neg_ids.astype(i32),
        h.astype(i32), r.astype(i32), pos_t.astype(i32), neg_t.astype(i32),
        user_table.reshape(-1, 4 * _D), item_table.reshape(-1, 4 * _D),
        entity_table.reshape(-1, 4 * _D), relation_table.reshape(-1, 4 * _D))
    s = jnp.sum(parts, axis=(0, 2))
    (a_h, a_r, a_p, a_n, c_hr, c_hp, c_hn, c_rp, c_rn,
     s_log, s_u, s_p, s_n) = [s[k] for k in range(13)]
    bf = float(_B)
    pos2 = a_h + a_r + a_p + 2.0 * (c_hr - c_hp - c_rp)
    neg2 = a_h + a_r + a_n + 2.0 * (c_hr - c_hn - c_rn)
    pos_s = jnp.sqrt(jnp.maximum(pos2, 0.0))
    neg_s = jnp.sqrt(jnp.maximum(neg2, 0.0))
    kg_total = (-jax.nn.log_sigmoid(pos_s - neg_s)
                + 1e-5 * (a_h + a_r + a_p + a_n) / (2.0 * bf))
    cf_total = s_log / bf + 1e-5 * (s_u + s_p + s_n) / (2.0 * bf)
    return kg_total + cf_total


# speedup vs baseline: 1.0144x; 1.0144x over previous
import functools

import jax
import jax.numpy as jnp
from jax import lax
from jax.experimental import pallas as pl
from jax.experimental.pallas import tpu as pltpu
from jax.experimental.pallas import tpu_sc as plsc

_B = 16384
_D = 32
_LN2 = 0.6931471805599453


def _rsqrt16(x):
    i = lax.bitcast_convert_type(x, jnp.int32)
    one = jnp.full((16,), 1, jnp.int32)
    y = lax.bitcast_convert_type(
        jnp.full((16,), 0x5F3759DF, jnp.int32) - lax.shift_right_logical(i, one),
        jnp.float32)
    for _ in range(3):
        y = y * (1.5 - 0.5 * x * y * y)
    return y


def _ln16(x):
    f = lax.bitcast_convert_type(x, jnp.int32).astype(jnp.float32)
    y = f * (_LN2 / (1 << 23)) - (127.0 * _LN2 + 0.0298)
    for _ in range(3):
        y = y + x * jnp.exp(-y) - 1.0
    return y


def _sc_loss_parts(uid, ipid, inid, hid, rid, ptid, ntid, ut, it, et, rt):
    info = plsc.get_sparse_core_info()
    nc, ns = info.num_cores, info.num_subcores
    nw = nc * ns
    rpw = _B // nw
    ngrp = rpw // 16

    mesh = plsc.VectorSubcoreMesh(core_axis_name="c", subcore_axis_name="s")

    @functools.partial(
        pl.kernel,
        mesh=mesh,
        compiler_params=pltpu.CompilerParams(use_tc_tiling_on_sc=False, needs_layout_passes=False),
        out_type=jax.ShapeDtypeStruct((nw, 16, 16), jnp.float32),
        scratch_types=(
            [pltpu.VMEM((rpw,), jnp.int32)] * 7
            + [pltpu.VMEM((rpw, _D), jnp.float32)] * 5
            + [pltpu.VMEM((16, 16), jnp.float32), pltpu.SemaphoreType.DMA]
        ),
    )
    def body(uid_h, ipid_h, inid_h, hid_h, rid_h, ptid_h, ntid_h,
             ut_h, it_h, et_h, rt_h, out_h,
             xu, xip, xin, xh, xr, xpt, xnt,
             b0, b1, b2, b3, b4, accv, sem):
        wid = lax.axis_index("s") * nc + lax.axis_index("c")
        sl = pl.ds(wid * rpw, rpw)
        pltpu.sync_copy(hid_h.at[sl], xh)
        pltpu.sync_copy(rid_h.at[sl], xr)
        pltpu.sync_copy(ptid_h.at[sl], xpt)
        pltpu.sync_copy(ntid_h.at[sl], xnt)
        pltpu.sync_copy(uid_h.at[sl], xu)
        pltpu.sync_copy(ipid_h.at[sl], xip)
        pltpu.sync_copy(inid_h.at[sl], xin)

        ch = pltpu.async_copy(et_h.at[xh], b0, sem)
        cr = pltpu.async_copy(rt_h.at[xr], b1, sem)
        cp = pltpu.async_copy(et_h.at[xpt], b2, sem)
        cn = pltpu.async_copy(et_h.at[xnt], b3, sem)
        ch.wait(); cr.wait(); cp.wait(); cn.wait()

        iota = lax.iota(jnp.int32, 16)
        zero = jnp.zeros((16,), jnp.float32)

        def kg_group(g, carry):
            a_h, a_r, a_p, a_n, c_hr, c_hp, c_hn, c_rp, c_rn = carry
            row = g * 16 + iota
            sh = sr = sp = sn = hr = hp = hn = rp = rn = zero
            for d in range(_D):
                col = jnp.full((16,), d, jnp.int32)
                hv = plsc.load_gather(b0, [row, col])
                rv = plsc.load_gather(b1, [row, col])
                pv = plsc.load_gather(b2, [row, col])
                nv = plsc.load_gather(b3, [row, col])
                sh = sh + hv * hv
                sr = sr + rv * rv
                sp = sp + pv * pv
                sn = sn + nv * nv
                hr = hr + hv * rv
                hp = hp + hv * pv
                hn = hn + hv * nv
                rp = rp + rv * pv
                rn = rn + rv * nv
            ih = _rsqrt16(jnp.maximum(sh, 1e-24))
            ir = _rsqrt16(jnp.maximum(sr, 1e-24))
            ip_ = _rsqrt16(jnp.maximum(sp, 1e-24))
            in_ = _rsqrt16(jnp.maximum(sn, 1e-24))
            return (a_h + sh * ih * ih, a_r + sr * ir * ir,
                    a_p + sp * ip_ * ip_, a_n + sn * in_ * in_,
                    c_hr + hr * ih * ir, c_hp + hp * ih * ip_,
                    c_hn + hn * ih * in_, c_rp + rp * ir * ip_,
                    c_rn + rn * ir * in_)

        kg = lax.fori_loop(0, ngrp, kg_group, (zero,) * 9)

        cu = pltpu.async_copy(ut_h.at[xu], b0, sem)
        cip = pltpu.async_copy(it_h.at[xip], b1, sem)
        cep = pltpu.async_copy(et_h.at[xip], b2, sem)
        cin = pltpu.async_copy(it_h.at[xin], b3, sem)
        cen = pltpu.async_copy(et_h.at[xin], b4, sem)
        cu.wait(); cip.wait(); cep.wait(); cin.wait(); cen.wait()

        def cf_group(g, carry):
            slog, su, sp2, sn2 = carry
            row = g * 16 + iota
            up = un = ru = rp_ = rn_ = zero
            for d in range(_D):
                col = jnp.full((16,), d, jnp.int32)
                uv = plsc.load_gather(b0, [row, col])
                ipv = plsc.load_gather(b1, [row, col])
                epv = plsc.load_gather(b2, [row, col])
                inv = plsc.load_gather(b3, [row, col])
                env = plsc.load_gather(b4, [row, col])
                pv = ipv + epv
                nv = inv + env
                up = up + uv * pv
                un = un + uv * nv
                ru = ru + uv * uv
                rp_ = rp_ + pv * pv
                rn_ = rn_ + nv * nv
            dlt = up - un
            sig = 1.0 / (1.0 + jnp.exp(-dlt))
            y = _ln16(sig + 1e-10)
            return (slog - y, su + ru, sp2 + rp_, sn2 + rn_)

        cf = lax.fori_loop(0, ngrp, cf_group, (zero,) * 4)

        vals = list(kg) + list(cf)
        for k in range(13):
            accv[k, :] = vals[k]
        for k in range(13, 16):
            accv[k, :] = zero
        pltpu.sync_copy(accv, out_h.at[wid])

    return body(uid, ipid, inid, hid, rid, ptid, ntid, ut, it, et, rt)


def kernel(user_ids, item_pos_ids, item_neg_ids, h, r, pos_t, neg_t, is_train,
           user_table, item_table, entity_table, relation_table):
    del is_train
    i32 = jnp.int32
    parts = _sc_loss_parts(
        user_ids.astype(i32), item_pos_ids.astype(i32), item_neg_ids.astype(i32),
        h.astype(i32), r.astype(i32), pos_t.astype(i32), neg_t.astype(i32),
        user_table, item_table, entity_table, relation_table)
    s = jnp.sum(parts, axis=(0, 2))
    (a_h, a_r, a_p, a_n, c_hr, c_hp, c_hn, c_rp, c_rn,
     s_log, s_u, s_p, s_n) = [s[k] for k in range(13)]
    bf = float(_B)
    pos2 = a_h + a_r + a_p + 2.0 * (c_hr - c_hp - c_rp)
    neg2 = a_h + a_r + a_n + 2.0 * (c_hr - c_hn - c_rn)
    pos_s = jnp.sqrt(jnp.maximum(pos2, 0.0))
    neg_s = jnp.sqrt(jnp.maximum(neg2, 0.0))
    kg_total = (-jax.nn.log_sigmoid(pos_s - neg_s)
                + 1e-5 * (a_h + a_r + a_p + a_n) / (2.0 * bf))
    cf_total = s_log / bf + 1e-5 * (s_u + s_p + s_n) / (2.0 * bf)
    return kg_total + cf_total
